# baseline (speedup 1.0000x reference)
import functools

import jax
import jax.numpy as jnp
from jax import lax
from jax.experimental import pallas as pl
from jax.experimental.pallas import tpu as pltpu
from jax.experimental.pallas import tpu_sc as plsc


def _gather_rows(table, idx):
    n, h = table.shape
    b = idx.shape[0]
    info = plsc.get_sparse_core_info()
    nw = info.num_cores * info.num_subcores
    bpw = b // nw
    ch = 8
    nch = bpw // ch
    mesh = plsc.VectorSubcoreMesh(core_axis_name="c", subcore_axis_name="s")

    @functools.partial(
        pl.kernel,
        mesh=mesh,
        out_type=jax.ShapeDtypeStruct((b, h), table.dtype),
        scratch_types=[
            pltpu.VMEM((bpw,), jnp.int32),
            pltpu.VMEM((ch, h), table.dtype),
            pltpu.VMEM((ch, h), table.dtype),
            pltpu.SemaphoreType.DMA,
            pltpu.SemaphoreType.DMA,
        ],
    )
    def k(table_hbm, idx_hbm, out_hbm, idx_v, buf0, buf1, sem0, sem1):
        wid = lax.axis_index("s") * info.num_cores + lax.axis_index("c")
        base = wid * bpw
        pltpu.sync_copy(idx_hbm.at[pl.ds(base, bpw)], idx_v)
        bufs = (buf0, buf1)
        sems = (sem0, sem1)

        def fire(c):
            return pltpu.async_copy(
                table_hbm.at[idx_v.at[pl.ds(c * ch, ch)]], bufs[c % 2], sems[c % 2]
            )

        cps = [fire(0), fire(1) if nch > 1 else None]
        for c in range(nch):
            cps[c % 2].wait()
            pltpu.sync_copy(bufs[c % 2], out_hbm.at[pl.ds(base + c * ch, ch)])
            if c + 2 < nch:
                cps[c % 2] = fire(c + 2)

    return k(table, idx)


def _ffn_body(st_ref, se_ref, lo_ref, hi_ref, xs_ref, w_ref, dw_ref, up_ref, out_ref):
    g = pl.program_id(0)
    t = xs_ref.shape[0]
    lo = lo_ref[g]
    hi = hi_ref[g]
    gm1 = jnp.maximum(g - 1, 0)
    first = jnp.logical_or(g == 0, st_ref[g] != st_ref[gm1])

    @pl.when(first)
    def _init():
        out_ref[...] = xs_ref[...]

    @pl.when(hi > lo)
    def _acc():
        xb = xs_ref[...].astype(jnp.bfloat16)
        dw = dw_ref[0].astype(jnp.bfloat16)
        h = jnp.dot(xb, dw, preferred_element_type=jnp.float32)
        h = jnp.maximum(h, 0.0).astype(jnp.bfloat16)
        up = up_ref[0].astype(jnp.bfloat16)
        y = jnp.dot(h, up, preferred_element_type=jnp.float32)
        rows = lax.broadcasted_iota(jnp.int32, (t, 1), 0)
        mask = jnp.logical_and(rows >= lo, rows < hi)
        out_ref[...] += jnp.where(mask, y * w_ref[...], 0.0)


def _grouped_ffn(xs, ws, down_proj, up_proj, step_tile, step_exp, lo, hi, tile_rows):
    s, hdim = xs.shape
    e, _, d = down_proj.shape
    g = step_tile.shape[0]
    grid_spec = pltpu.PrefetchScalarGridSpec(
        num_scalar_prefetch=4,
        grid=(g,),
        in_specs=[
            pl.BlockSpec((tile_rows, hdim), lambda i, st, se, lo, hi: (st[i], 0)),
            pl.BlockSpec((tile_rows, 1), lambda i, st, se, lo, hi: (st[i], 0)),
            pl.BlockSpec((1, hdim, d), lambda i, st, se, lo, hi: (se[i], 0, 0)),
            pl.BlockSpec((1, d, hdim), lambda i, st, se, lo, hi: (se[i], 0, 0)),
        ],
        out_specs=pl.BlockSpec((tile_rows, hdim), lambda i, st, se, lo, hi: (st[i], 0)),
    )
    return pl.pallas_call(
        _ffn_body,
        grid_spec=grid_spec,
        out_shape=jax.ShapeDtypeStruct((s, hdim), jnp.float32),
    )(step_tile, step_exp, lo, hi, xs, ws, down_proj, up_proj)


def kernel(x, attention_mask, expert_weights, chosen_expert_indices, down_proj, up_proj):
    b_, s_, hdim = x.shape
    e = down_proj.shape[0]
    s = b_ * s_
    tile_rows = 512
    nt = s // tile_rows
    g = nt + e

    xf = x.reshape(s, hdim)
    e_ids = chosen_expert_indices.reshape(s).astype(jnp.int32)
    w_eff = expert_weights.reshape(s) * attention_mask.reshape(s)

    iota_s = jnp.arange(s, dtype=jnp.int32)
    sorted_e, perm, w_sorted_flat = lax.sort(
        (e_ids, iota_s, w_eff), dimension=0, num_keys=1
    )
    inv_perm = jnp.zeros(s, jnp.int32).at[perm].set(iota_s)
    w_sorted = w_sorted_flat.reshape(s, 1)
    prev_e = jnp.concatenate([sorted_e[:1] - 1, sorted_e[:-1]])
    marks = jnp.logical_or(iota_s % tile_rows == 0, sorted_e != prev_e)
    steps_r = jnp.nonzero(marks, size=g, fill_value=s)[0].astype(jnp.int32)
    step_tile = jnp.minimum(steps_r // tile_rows, nt - 1)
    step_exp = jnp.take(sorted_e, jnp.minimum(steps_r, s - 1))
    r_next = jnp.concatenate([steps_r[1:], jnp.full((1,), s, jnp.int32)])
    lo = steps_r - step_tile * tile_rows
    hi = jnp.minimum((step_tile + 1) * tile_rows, r_next) - step_tile * tile_rows

    xs = _gather_rows(xf, perm.astype(jnp.int32))
    ys = _grouped_ffn(
        xs, w_sorted, down_proj, up_proj, step_tile, step_exp, lo, hi, tile_rows
    )
    out = _gather_rows(ys, inv_perm.astype(jnp.int32))
    return out.reshape(b_, s_, hdim)


# speedup vs baseline: 1.0877x; 1.0877x over previous
import functools

import jax
import jax.numpy as jnp
from jax import lax
from jax.experimental import pallas as pl
from jax.experimental.pallas import tpu as pltpu
from jax.experimental.pallas import tpu_sc as plsc


def _gather_rows(table, idx):
    n, h = table.shape
    b = idx.shape[0]
    info = plsc.get_sparse_core_info()
    nw = info.num_cores * info.num_subcores
    bpw = b // nw
    ch = 16
    nch = bpw // ch
    mesh = plsc.VectorSubcoreMesh(core_axis_name="c", subcore_axis_name="s")

    @functools.partial(
        pl.kernel,
        mesh=mesh,
        out_type=jax.ShapeDtypeStruct((b, h), table.dtype),
        scratch_types=[
            pltpu.VMEM((bpw,), jnp.int32),
            pltpu.VMEM((ch, h), table.dtype),
            pltpu.VMEM((ch, h), table.dtype),
            pltpu.SemaphoreType.DMA,
            pltpu.SemaphoreType.DMA,
        ],
    )
    def k(table_hbm, idx_hbm, out_hbm, idx_v, buf0, buf1, sem0, sem1):
        wid = lax.axis_index("s") * info.num_cores + lax.axis_index("c")
        base = wid * bpw
        pltpu.sync_copy(idx_hbm.at[pl.ds(base, bpw)], idx_v)
        bufs = (buf0, buf1)
        sems = (sem0, sem1)

        def fire(c):
            return pltpu.async_copy(
                table_hbm.at[idx_v.at[pl.ds(c * ch, ch)]], bufs[c % 2], sems[c % 2]
            )

        cps = [fire(0), fire(1) if nch > 1 else None]
        for c in range(nch):
            cps[c % 2].wait()
            pltpu.sync_copy(bufs[c % 2], out_hbm.at[pl.ds(base + c * ch, ch)])
            if c + 2 < nch:
                cps[c % 2] = fire(c + 2)

    return k(table, idx)


def _ffn_body(st_ref, se_ref, lo_ref, hi_ref, xs_ref, w_ref, dw_ref, up_ref, out_ref):
    g = pl.program_id(0)
    t = xs_ref.shape[0]
    lo = lo_ref[g]
    hi = hi_ref[g]
    gm1 = jnp.maximum(g - 1, 0)
    first = jnp.logical_or(g == 0, st_ref[g] != st_ref[gm1])

    @pl.when(first)
    def _init():
        out_ref[...] = xs_ref[...]

    @pl.when(hi > lo)
    def _acc():
        xb = xs_ref[...].astype(jnp.bfloat16)
        dw = dw_ref[0].astype(jnp.bfloat16)
        h = jnp.dot(xb, dw, preferred_element_type=jnp.float32)
        h = jnp.maximum(h, 0.0).astype(jnp.bfloat16)
        up = up_ref[0].astype(jnp.bfloat16)
        y = jnp.dot(h, up, preferred_element_type=jnp.float32)
        rows = lax.broadcasted_iota(jnp.int32, (t, 1), 0)
        mask = jnp.logical_and(rows >= lo, rows < hi)
        out_ref[...] += jnp.where(mask, y * w_ref[...], 0.0)


def _grouped_ffn(xs, ws, down_proj, up_proj, step_tile, step_exp, lo, hi, tile_rows):
    s, hdim = xs.shape
    e, _, d = down_proj.shape
    g = step_tile.shape[0]
    grid_spec = pltpu.PrefetchScalarGridSpec(
        num_scalar_prefetch=4,
        grid=(g,),
        in_specs=[
            pl.BlockSpec((tile_rows, hdim), lambda i, st, se, lo, hi: (st[i], 0)),
            pl.BlockSpec((tile_rows, 1), lambda i, st, se, lo, hi: (st[i], 0)),
            pl.BlockSpec((1, hdim, d), lambda i, st, se, lo, hi: (se[i], 0, 0)),
            pl.BlockSpec((1, d, hdim), lambda i, st, se, lo, hi: (se[i], 0, 0)),
        ],
        out_specs=pl.BlockSpec((tile_rows, hdim), lambda i, st, se, lo, hi: (st[i], 0)),
    )
    return pl.pallas_call(
        _ffn_body,
        grid_spec=grid_spec,
        out_shape=jax.ShapeDtypeStruct((s, hdim), jnp.float32),
    )(step_tile, step_exp, lo, hi, xs, ws, down_proj, up_proj)


def kernel(x, attention_mask, expert_weights, chosen_expert_indices, down_proj, up_proj):
    b_, s_, hdim = x.shape
    e = down_proj.shape[0]
    s = b_ * s_
    tile_rows = 256
    nt = s // tile_rows
    g = nt + e

    xf = x.reshape(s, hdim)
    e_ids = chosen_expert_indices.reshape(s).astype(jnp.int32)
    w_eff = expert_weights.reshape(s) * attention_mask.reshape(s)

    iota_s = jnp.arange(s, dtype=jnp.int32)
    sorted_e, perm, w_sorted_flat = lax.sort(
        (e_ids, iota_s, w_eff), dimension=0, num_keys=1
    )
    inv_perm = jnp.zeros(s, jnp.int32).at[perm].set(iota_s)
    w_sorted = w_sorted_flat.reshape(s, 1)
    prev_e = jnp.concatenate([sorted_e[:1] - 1, sorted_e[:-1]])
    marks = jnp.logical_or(iota_s % tile_rows == 0, sorted_e != prev_e)
    steps_r = jnp.nonzero(marks, size=g, fill_value=s)[0].astype(jnp.int32)
    step_tile = jnp.minimum(steps_r // tile_rows, nt - 1)
    step_exp = jnp.take(sorted_e, jnp.minimum(steps_r, s - 1))
    r_next = jnp.concatenate([steps_r[1:], jnp.full((1,), s, jnp.int32)])
    lo = steps_r - step_tile * tile_rows
    hi = jnp.minimum((step_tile + 1) * tile_rows, r_next) - step_tile * tile_rows

    xs = _gather_rows(xf, perm.astype(jnp.int32))
    ys = _grouped_ffn(
        xs, w_sorted, down_proj, up_proj, step_tile, step_exp, lo, hi, tile_rows
    )
    out = _gather_rows(ys, inv_perm.astype(jnp.int32))
    return out.reshape(b_, s_, hdim)


# speedup vs baseline: 1.1070x; 1.0178x over previous
import functools

import jax
import jax.numpy as jnp
from jax import lax
from jax.experimental import pallas as pl
from jax.experimental.pallas import tpu as pltpu
from jax.experimental.pallas import tpu_sc as plsc


def _gather_rows(table, idx):
    n, h = table.shape
    b = idx.shape[0]
    info = plsc.get_sparse_core_info()
    nw = info.num_cores * info.num_subcores
    bpw = b // nw
    ch = 16
    nch = bpw // ch
    mesh = plsc.VectorSubcoreMesh(core_axis_name="c", subcore_axis_name="s")

    @functools.partial(
        pl.kernel,
        mesh=mesh,
        out_type=jax.ShapeDtypeStruct((b, h), table.dtype),
        scratch_types=[
            pltpu.VMEM((bpw,), jnp.int32),
            pltpu.VMEM((ch, h), table.dtype),
            pltpu.VMEM((ch, h), table.dtype),
            pltpu.SemaphoreType.DMA,
            pltpu.SemaphoreType.DMA,
        ],
    )
    def k(table_hbm, idx_hbm, out_hbm, idx_v, buf0, buf1, sem0, sem1):
        wid = lax.axis_index("s") * info.num_cores + lax.axis_index("c")
        base = wid * bpw
        pltpu.sync_copy(idx_hbm.at[pl.ds(base, bpw)], idx_v)
        bufs = (buf0, buf1)
        sems = (sem0, sem1)

        def fire(c):
            return pltpu.async_copy(
                table_hbm.at[idx_v.at[pl.ds(c * ch, ch)]], bufs[c % 2], sems[c % 2]
            )

        cps = [fire(0), fire(1) if nch > 1 else None]
        for c in range(nch):
            cps[c % 2].wait()
            pltpu.sync_copy(bufs[c % 2], out_hbm.at[pl.ds(base + c * ch, ch)])
            if c + 2 < nch:
                cps[c % 2] = fire(c + 2)

    return k(table, idx)


def _grouped_ffn(xs, ws, down_proj, up_proj, steps_r, steps_e, tile_rows):
    s, hdim = xs.shape
    e, _, d = down_proj.shape
    g = steps_r.shape[0]
    nt = s // tile_rows

    def row_tile(r):
        return jnp.minimum(lax.div(r, tile_rows), nt - 1)

    def body(sr_ref, se_ref, xs_ref, w_ref, dw_ref, up_ref, out_ref):
        i = pl.program_id(0)
        r = sr_ref[i]
        tile = row_tile(r)
        lo = r - tile * tile_rows
        rn = sr_ref[jnp.minimum(i + 1, g - 1)]
        hi = jnp.minimum((tile + 1) * tile_rows, rn) - tile * tile_rows
        rm1 = sr_ref[jnp.maximum(i - 1, 0)]
        first = jnp.logical_or(i == 0, tile != row_tile(rm1))

        @pl.when(first)
        def _init():
            out_ref[...] = xs_ref[...]

        @pl.when(hi > lo)
        def _acc():
            xb = xs_ref[...].astype(jnp.bfloat16)
            dw = dw_ref[0].astype(jnp.bfloat16)
            h = jnp.dot(xb, dw, preferred_element_type=jnp.float32)
            h = jnp.maximum(h, 0.0).astype(jnp.bfloat16)
            up = up_ref[0].astype(jnp.bfloat16)
            y = jnp.dot(h, up, preferred_element_type=jnp.float32)
            rows = lax.broadcasted_iota(jnp.int32, (tile_rows, 1), 0)
            mask = jnp.logical_and(rows >= lo, rows < hi)
            out_ref[...] += jnp.where(mask, y * w_ref[...], 0.0)

    grid_spec = pltpu.PrefetchScalarGridSpec(
        num_scalar_prefetch=2,
        grid=(g,),
        in_specs=[
            pl.BlockSpec((tile_rows, hdim), lambda i, sr, se: (row_tile(sr[i]), 0)),
            pl.BlockSpec((tile_rows, 1), lambda i, sr, se: (row_tile(sr[i]), 0)),
            pl.BlockSpec((1, hdim, d), lambda i, sr, se: (se[i], 0, 0)),
            pl.BlockSpec((1, d, hdim), lambda i, sr, se: (se[i], 0, 0)),
        ],
        out_specs=pl.BlockSpec(
            (tile_rows, hdim), lambda i, sr, se: (row_tile(sr[i]), 0)
        ),
    )
    return pl.pallas_call(
        body,
        grid_spec=grid_spec,
        out_shape=jax.ShapeDtypeStruct((s, hdim), jnp.float32),
    )(steps_r, steps_e, xs, ws, down_proj, up_proj)


def kernel(x, attention_mask, expert_weights, chosen_expert_indices, down_proj, up_proj):
    b_, s_, hdim = x.shape
    e = down_proj.shape[0]
    s = b_ * s_
    tile_rows = 256
    nt = s // tile_rows
    g = -(-(nt + e) // 16) * 16

    xf = x.reshape(s, hdim)
    e_ids = chosen_expert_indices.reshape(s).astype(jnp.int32)
    w_eff = expert_weights.reshape(s) * attention_mask.reshape(s)

    iota_s = jnp.arange(s, dtype=jnp.int32)
    sorted_e, perm, w_sorted_flat = lax.sort(
        (e_ids, iota_s, w_eff), dimension=0, num_keys=1
    )
    inv_perm = jnp.zeros(s, jnp.int32).at[perm].set(iota_s)
    w_sorted = w_sorted_flat.reshape(s, 1)

    prev_e = jnp.concatenate([sorted_e[:1] - 1, sorted_e[:-1]])
    unmarked = jnp.where(
        jnp.logical_or(iota_s % tile_rows == 0, sorted_e != prev_e), 0, 1
    ).astype(jnp.int32)
    um_s, sr_full, se_full = lax.sort(
        (unmarked, iota_s, sorted_e), dimension=0, num_keys=1, is_stable=True
    )
    keep = um_s[:g] == 0
    steps_r = jnp.where(keep, sr_full[:g], s)
    steps_e = jnp.where(keep, se_full[:g], e - 1)

    xs = _gather_rows(xf, perm.astype(jnp.int32))
    ys = _grouped_ffn(xs, w_sorted, down_proj, up_proj, steps_r, steps_e, tile_rows)
    out = _gather_rows(ys, inv_perm.astype(jnp.int32))
    return out.reshape(b_, s_, hdim)
